# baseline (speedup 1.0000x reference)
import functools

import jax
import jax.numpy as jnp
from jax import lax
from jax.experimental import pallas as pl
from jax.experimental.pallas import tpu as pltpu
from jax.experimental.pallas import tpu_sc as plsc

_EMBED = 128
_BATCH = 16384
_MARGIN = 1.0
_LANES = 16
_NSEG = _EMBED // _LANES

_NC = 2
_NS = 16
_NW = _NC * _NS
_B_PER_W = _BATCH // _NW
_CHUNK = 32
_NCHUNK = _B_PER_W // _CHUNK
_NSET = 3


def _tec_kernel(pos_hbm, neg_hbm, ent_hbm, rel_hbm, out_hbm, *refs):
    idx_refs = refs[0:6]
    bufsets = tuple(tuple(refs[6 + 6 * s: 12 + 6 * s]) for s in range(_NSET))
    out_v = refs[6 + 6 * _NSET]
    sems = refs[7 + 6 * _NSET: 7 + 7 * _NSET]

    wid = lax.axis_index("s") * _NC + lax.axis_index("c")
    base0 = wid * _B_PER_W
    sl0 = pl.ds(base0, _B_PER_W)

    pltpu.sync_copy(pos_hbm.at[pl.ds(0, 1), sl0], idx_refs[0])
    pltpu.sync_copy(pos_hbm.at[pl.ds(1, 1), sl0], idx_refs[1])
    pltpu.sync_copy(pos_hbm.at[pl.ds(2, 1), sl0], idx_refs[2])
    pltpu.sync_copy(neg_hbm.at[pl.ds(0, 1), sl0], idx_refs[3])
    pltpu.sync_copy(neg_hbm.at[pl.ds(1, 1), sl0], idx_refs[4])
    pltpu.sync_copy(neg_hbm.at[pl.ds(2, 1), sl0], idx_refs[5])

    tables = (ent_hbm, rel_hbm, ent_hbm, ent_hbm, rel_hbm, ent_hbm)

    def fire(ci):
        s = ci % _NSET
        return [pltpu.async_copy(
                    tab.at[idx.at[0, pl.ds(ci * _CHUNK, _CHUNK)]], buf,
                    sems[s])
                for tab, idx, buf in zip(tables, idx_refs, bufsets[s])]

    lane = lax.broadcasted_iota(jnp.int32, (_LANES,), 0)

    def compute_chunk(s, acc0):
        ph, pr, pt, nh, nr, nt = bufsets[s]

        def row_body(b, acc):
            d = jnp.zeros((_LANES,), jnp.float32)
            for j in range(_NSEG):
                ds = pl.ds(j * _LANES, _LANES)
                pd = jnp.abs(ph[b, ds] + pr[b, ds] - pt[b, ds])
                nd = jnp.abs(nh[b, ds] + nr[b, ds] - nt[b, ds])
                d = d + (nd - pd)
            for k in (1, 2, 4, 8):
                d = d + d.at[lane ^ k].get(mode="promise_in_bounds")
            c = jnp.maximum(d + _MARGIN, 0.0)
            return acc + jnp.where(lane == 0, c, 0.0)

        return lax.fori_loop(0, _CHUNK, row_body, acc0)

    acc = jnp.zeros((_LANES,), jnp.float32)
    pend = [fire(0), fire(1)]
    for ci in range(_NCHUNK):
        if ci + 2 < _NCHUNK:
            pend.append(fire(ci + 2))
        for cp in pend.pop(0):
            cp.wait()
        acc = compute_chunk(ci % _NSET, acc)

    out_v[...] = acc
    pltpu.sync_copy(out_v, out_hbm.at[wid])


@jax.jit
def kernel(pos_exmpl, neg_exmpl, entity_emb, relation_emb):
    mesh = plsc.VectorSubcoreMesh(core_axis_name="c", subcore_axis_name="s")
    buf = pltpu.VMEM((_CHUNK, _EMBED), jnp.float32)
    run = functools.partial(
        pl.kernel,
        mesh=mesh,
        out_type=jax.ShapeDtypeStruct((_NW, _LANES), jnp.float32),
        scratch_types=(
            [pltpu.VMEM((1, _B_PER_W), jnp.int32)] * 6
            + [buf] * (6 * _NSET)
            + [pltpu.VMEM((_LANES,), jnp.float32)]
            + [pltpu.SemaphoreType.DMA] * _NSET
        ),
    )(_tec_kernel)
    partials = run(pos_exmpl, neg_exmpl, entity_emb, relation_emb)
    return jnp.sum(partials)
